# baseline (speedup 1.0000x reference)
import functools

import jax
import jax.numpy as jnp
from jax.experimental import pallas as pl
from jax.experimental.pallas import tpu as pltpu

M = 64
D_IN = 1024
D_OUT = 1024
TOKENS = 8192
TOPK = 8
TAU = 0.1
BETA = 10.0
THETA = 0.5
GAMMA = 0.5

BLOCK = 2048
NB = TOKENS // BLOCK
NEG = -1e30


def _body(x_ref, k_ref, v_ref, s_ref, y_ref, attn_ref, f_ref, g_ref,
          s_scr, acc_scr, cnt_scr):
    p = pl.program_id(0)
    i = pl.program_id(1)

    @pl.when(p == 0)
    def phase0():
        xb = x_ref[...]
        xn = xb * (1.0 / (jnp.sqrt(jnp.sum(xb * xb, axis=1, keepdims=True))
                          + 1e-12))
        kb = k_ref[...]
        kn = kb * (1.0 / (jnp.sqrt(jnp.sum(kb * kb, axis=1, keepdims=True))
                          + 1e-12))
        sb = jax.lax.dot_general(xn, kn, (((1,), (1,)), ((), ())),
                                 preferred_element_type=jnp.float32)
        s_scr[pl.ds(i * BLOCK, BLOCK), :] = sb

        m = jnp.max(sb, axis=1, keepdims=True)
        eq = (sb == m).astype(jnp.float32)
        rr = jax.lax.broadcasted_iota(jnp.int32, (M, M), 0)
        cc = jax.lax.broadcasted_iota(jnp.int32, (M, M), 1)
        lt = (rr < cc).astype(jnp.float32)
        pre = jax.lax.dot_general(eq, lt, (((1,), (0,)), ((), ())),
                                  preferred_element_type=jnp.float32)
        onehot = jnp.where(pre > 0.0, 0.0, eq)

        @pl.when(i == 0)
        def _init():
            acc_scr[...] = onehot

        @pl.when(i != 0)
        def _acc():
            acc_scr[...] += onehot

    @pl.when(p == 1)
    def phase1():
        @pl.when(i == 0)
        def _reduce_counts():
            ones = jnp.ones((1, BLOCK), dtype=jnp.float32)
            cnt_scr[...] = jax.lax.dot_general(
                ones, acc_scr[...], (((1,), (0,)), ((), ())),
                preferred_element_type=jnp.float32)

        sb = s_scr[pl.ds(i * BLOCK, BLOCK), :]

        rr = jax.lax.broadcasted_iota(jnp.int32, (M, M), 0)
        cc = jax.lax.broadcasted_iota(jnp.int32, (M, M), 1)
        lt = (rr < cc).astype(jnp.float32)
        work = sb
        mask = jnp.zeros((BLOCK, M), dtype=jnp.bool_)
        f = None
        for t in range(TOPK):
            mx = jnp.max(work, axis=1, keepdims=True)
            if t == 0:
                f = mx
            eq = (work == mx).astype(jnp.float32)
            pre = jax.lax.dot_general(eq, lt, (((1,), (0,)), ((), ())),
                                      preferred_element_type=jnp.float32)
            sel = jnp.logical_and(pre == 0.0, eq > 0.0)
            mask = jnp.logical_or(mask, sel)
            work = jnp.where(sel, NEG, work)

        counts = cnt_scr[...]
        bias = (-GAMMA / TOKENS) * counts
        ls = jnp.log(s_ref[...] + 1e-08) + bias
        ub = f * (1.0 / TAU) + jnp.max(ls)
        e = jnp.where(mask, jnp.exp(sb * (1.0 / TAU) + ls - ub), 0.0)
        attn = e * (1.0 / jnp.sum(e, axis=1, keepdims=True))

        vr = jax.lax.dot_general(attn, v_ref[...], (((1,), (0,)), ((), ())),
                                 preferred_element_type=jnp.float32)
        g = 1.0 - jax.nn.sigmoid(BETA * (f - THETA))

        y_ref[...] = g * vr
        attn_ref[...] = attn
        f_ref[...] = f
        g_ref[...] = g


@jax.jit
def kernel(x, K, V, s):
    s2 = s.reshape(1, M)
    out_shapes = (
        jax.ShapeDtypeStruct((TOKENS, D_OUT), jnp.float32),
        jax.ShapeDtypeStruct((TOKENS, M), jnp.float32),
        jax.ShapeDtypeStruct((TOKENS, 1), jnp.float32),
        jax.ShapeDtypeStruct((TOKENS, 1), jnp.float32),
    )
    in_specs = [
        pl.BlockSpec((BLOCK, D_IN),
                     lambda p, i: (jax.lax.select(p == 0, i, NB - 1), 0)),
        pl.BlockSpec((M, D_IN), lambda p, i: (0, 0)),
        pl.BlockSpec((M, D_OUT), lambda p, i: (0, 0)),
        pl.BlockSpec((1, M), lambda p, i: (0, 0)),
    ]
    out_idx = lambda p, i: (jax.lax.select(p == 0, 0, i), 0)
    out_specs = (
        pl.BlockSpec((BLOCK, D_OUT), out_idx),
        pl.BlockSpec((BLOCK, M), out_idx),
        pl.BlockSpec((BLOCK, 1), out_idx),
        pl.BlockSpec((BLOCK, 1), out_idx),
    )
    y, attn, f, g = pl.pallas_call(
        _body,
        grid=(2, NB),
        in_specs=in_specs,
        out_specs=out_specs,
        out_shape=out_shapes,
        scratch_shapes=[
            pltpu.VMEM((TOKENS, M), jnp.float32),
            pltpu.VMEM((BLOCK, M), jnp.float32),
            pltpu.VMEM((1, M), jnp.float32),
        ],
        compiler_params=pltpu.CompilerParams(
            dimension_semantics=("arbitrary", "arbitrary"),
        ),
    )(x, K, V, s2)
    return (y, f.reshape(TOKENS), g.reshape(TOKENS), attn)


# speedup vs baseline: 1.0227x; 1.0227x over previous
import functools

import jax
import jax.numpy as jnp
from jax.experimental import pallas as pl
from jax.experimental.pallas import tpu as pltpu

M = 64
D_IN = 1024
D_OUT = 1024
TOKENS = 8192
TOPK = 8
TAU = 0.1
BETA = 10.0
THETA = 0.5
GAMMA = 0.5

BLOCK = 2048
NB = TOKENS // BLOCK
NEG = -1e30


def _body(x_ref, k_ref, v_ref, s_ref, y_ref, attn_ref, f_ref, g_ref,
          s_scr, acc_scr, cnt_scr):
    p = pl.program_id(0)
    i = pl.program_id(1)

    @pl.when(p == 0)
    def phase0():
        xb = x_ref[...]
        xn = xb * (1.0 / (jnp.sqrt(jnp.sum(xb * xb, axis=1, keepdims=True))
                          + 1e-12))
        kb = k_ref[...]
        kn = kb * (1.0 / (jnp.sqrt(jnp.sum(kb * kb, axis=1, keepdims=True))
                          + 1e-12))
        sb = jax.lax.dot_general(xn, kn, (((1,), (1,)), ((), ())),
                                 preferred_element_type=jnp.float32)
        s_scr[pl.ds(i * BLOCK, BLOCK), :] = sb

        m = jnp.max(sb, axis=1, keepdims=True)
        eq = (sb == m).astype(jnp.float32)
        rr = jax.lax.broadcasted_iota(jnp.int32, (M, M), 0)
        cc = jax.lax.broadcasted_iota(jnp.int32, (M, M), 1)
        lt = (rr < cc).astype(jnp.float32)
        pre = jax.lax.dot_general(eq, lt, (((1,), (0,)), ((), ())),
                                  preferred_element_type=jnp.float32)
        onehot = jnp.where(pre > 0.0, 0.0, eq)

        @pl.when(i == 0)
        def _init():
            acc_scr[...] = onehot

        @pl.when(i != 0)
        def _acc():
            acc_scr[...] += onehot

    @pl.when(p == 1)
    def phase1():
        @pl.when(i == 0)
        def _reduce_counts():
            ones = jnp.ones((1, BLOCK), dtype=jnp.float32)
            cnt_scr[...] = jax.lax.dot_general(
                ones, acc_scr[...], (((1,), (0,)), ((), ())),
                preferred_element_type=jnp.float32)

        sb = s_scr[pl.ds(i * BLOCK, BLOCK), :]

        rr = jax.lax.broadcasted_iota(jnp.int32, (M, M), 0)
        cc = jax.lax.broadcasted_iota(jnp.int32, (M, M), 1)
        lt = (rr < cc).astype(jnp.float32)
        work = sb
        mask = jnp.zeros((BLOCK, M), dtype=jnp.bool_)
        f = None
        for t in range(TOPK):
            mx = jnp.max(work, axis=1, keepdims=True)
            if t == 0:
                f = mx
            eq = (work == mx).astype(jnp.float32)
            pre = jax.lax.dot_general(eq, lt, (((1,), (0,)), ((), ())),
                                      preferred_element_type=jnp.float32)
            sel = jnp.logical_and(pre == 0.0, eq > 0.0)
            mask = jnp.logical_or(mask, sel)
            work = jnp.where(sel, NEG, work)

        counts = cnt_scr[...]
        bias = (-GAMMA / TOKENS) * counts
        logits = sb * (1.0 / TAU) + jnp.log(s_ref[...] + 1e-08) + bias
        logits = jnp.where(mask, logits, NEG)
        mrow = jnp.max(logits, axis=1, keepdims=True)
        e = jnp.where(mask, jnp.exp(logits - mrow), 0.0)
        attn = e * (1.0 / jnp.sum(e, axis=1, keepdims=True))

        vr = jax.lax.dot_general(attn, v_ref[...], (((1,), (0,)), ((), ())),
                                 preferred_element_type=jnp.float32)
        g = 1.0 - jax.nn.sigmoid(BETA * (f - THETA))

        y_ref[...] = g * vr
        attn_ref[...] = attn
        f_ref[...] = f
        g_ref[...] = g


@jax.jit
def kernel(x, K, V, s):
    s2 = s.reshape(1, M)
    out_shapes = (
        jax.ShapeDtypeStruct((TOKENS, D_OUT), jnp.float32),
        jax.ShapeDtypeStruct((TOKENS, M), jnp.float32),
        jax.ShapeDtypeStruct((TOKENS, 1), jnp.float32),
        jax.ShapeDtypeStruct((TOKENS, 1), jnp.float32),
    )
    in_specs = [
        pl.BlockSpec((BLOCK, D_IN),
                     lambda p, i: (jax.lax.select(p == 0, i, NB - 1), 0)),
        pl.BlockSpec((M, D_IN), lambda p, i: (0, 0)),
        pl.BlockSpec((M, D_OUT), lambda p, i: (0, 0)),
        pl.BlockSpec((1, M), lambda p, i: (0, 0)),
    ]
    out_idx = lambda p, i: (jax.lax.select(p == 0, 0, i), 0)
    out_specs = (
        pl.BlockSpec((BLOCK, D_OUT), out_idx),
        pl.BlockSpec((BLOCK, M), out_idx),
        pl.BlockSpec((BLOCK, 1), out_idx),
        pl.BlockSpec((BLOCK, 1), out_idx),
    )
    y, attn, f, g = pl.pallas_call(
        _body,
        grid=(2, NB),
        in_specs=in_specs,
        out_specs=out_specs,
        out_shape=out_shapes,
        scratch_shapes=[
            pltpu.VMEM((TOKENS, M), jnp.float32),
            pltpu.VMEM((BLOCK, M), jnp.float32),
            pltpu.VMEM((1, M), jnp.float32),
        ],
        compiler_params=pltpu.CompilerParams(
            dimension_semantics=("arbitrary", "arbitrary"),
        ),
    )(x, K, V, s2)
    return (y, f.reshape(TOKENS), g.reshape(TOKENS), attn)


# speedup vs baseline: 1.0332x; 1.0103x over previous
import functools

import jax
import jax.numpy as jnp
from jax.experimental import pallas as pl
from jax.experimental.pallas import tpu as pltpu

M = 64
D_IN = 1024
D_OUT = 1024
TOKENS = 8192
TOPK = 8
TAU = 0.1
BETA = 10.0
THETA = 0.5
GAMMA = 0.5

BLOCK = 2048
NB = TOKENS // BLOCK
NEG = -1e30


def _body(x_ref, k_ref, v_ref, s_ref, y_ref, attn_ref, f_ref, g_ref,
          s_scr, acc_scr, cnt_scr):
    p = pl.program_id(0)
    i = pl.program_id(1)

    @pl.when(p == 0)
    def phase0():
        xb = x_ref[...]
        xn = xb * (1.0 / (jnp.sqrt(jnp.sum(xb * xb, axis=1, keepdims=True))
                          + 1e-12))
        kb = k_ref[...]
        kn = kb * (1.0 / (jnp.sqrt(jnp.sum(kb * kb, axis=1, keepdims=True))
                          + 1e-12))
        sb = jax.lax.dot_general(xn, kn, (((1,), (1,)), ((), ())),
                                 preferred_element_type=jnp.float32)
        s_scr[pl.ds(i * BLOCK, BLOCK), :] = sb

        m = jnp.max(sb, axis=1, keepdims=True)
        eq = (sb == m).astype(jnp.float32)

        @pl.when(i == 0)
        def _init():
            acc_scr[...] = eq

        @pl.when(i != 0)
        def _acc():
            acc_scr[...] += eq

    @pl.when(p == 1)
    def phase1():
        @pl.when(i == 0)
        def _reduce_counts():
            ones = jnp.ones((1, BLOCK), dtype=jnp.float32)
            cnt_scr[...] = jax.lax.dot_general(
                ones, acc_scr[...], (((1,), (0,)), ((), ())),
                preferred_element_type=jnp.float32)

        sb = s_scr[pl.ds(i * BLOCK, BLOCK), :]

        rr = jax.lax.broadcasted_iota(jnp.int32, (M, M), 0)
        cc = jax.lax.broadcasted_iota(jnp.int32, (M, M), 1)
        lt = (rr < cc).astype(jnp.float32)
        work = sb
        mask = jnp.zeros((BLOCK, M), dtype=jnp.bool_)
        f = None
        for t in range(TOPK):
            mx = jnp.max(work, axis=1, keepdims=True)
            if t == 0:
                f = mx
            eq = (work == mx).astype(jnp.float32)
            pre = jax.lax.dot_general(eq, lt, (((1,), (0,)), ((), ())),
                                      preferred_element_type=jnp.float32)
            sel = jnp.logical_and(pre == 0.0, eq > 0.0)
            mask = jnp.logical_or(mask, sel)
            work = jnp.where(sel, NEG, work)

        counts = cnt_scr[...]
        bias = (-GAMMA / TOKENS) * counts
        logits = sb * (1.0 / TAU) + jnp.log(s_ref[...] + 1e-08) + bias
        logits = jnp.where(mask, logits, NEG)
        mrow = jnp.max(logits, axis=1, keepdims=True)
        e = jnp.where(mask, jnp.exp(logits - mrow), 0.0)
        attn = e * (1.0 / jnp.sum(e, axis=1, keepdims=True))

        vr = jax.lax.dot_general(attn, v_ref[...], (((1,), (0,)), ((), ())),
                                 preferred_element_type=jnp.float32)
        g = 1.0 - jax.nn.sigmoid(BETA * (f - THETA))

        y_ref[...] = g * vr
        attn_ref[...] = attn
        f_ref[...] = f
        g_ref[...] = g


@jax.jit
def kernel(x, K, V, s):
    s2 = s.reshape(1, M)
    out_shapes = (
        jax.ShapeDtypeStruct((TOKENS, D_OUT), jnp.float32),
        jax.ShapeDtypeStruct((TOKENS, M), jnp.float32),
        jax.ShapeDtypeStruct((TOKENS, 1), jnp.float32),
        jax.ShapeDtypeStruct((TOKENS, 1), jnp.float32),
    )
    in_specs = [
        pl.BlockSpec((BLOCK, D_IN),
                     lambda p, i: (jax.lax.select(p == 0, i, NB - 1), 0)),
        pl.BlockSpec((M, D_IN), lambda p, i: (0, 0)),
        pl.BlockSpec((M, D_OUT), lambda p, i: (0, 0)),
        pl.BlockSpec((1, M), lambda p, i: (0, 0)),
    ]
    out_idx = lambda p, i: (jax.lax.select(p == 0, 0, i), 0)
    out_specs = (
        pl.BlockSpec((BLOCK, D_OUT), out_idx),
        pl.BlockSpec((BLOCK, M), out_idx),
        pl.BlockSpec((BLOCK, 1), out_idx),
        pl.BlockSpec((BLOCK, 1), out_idx),
    )
    y, attn, f, g = pl.pallas_call(
        _body,
        grid=(2, NB),
        in_specs=in_specs,
        out_specs=out_specs,
        out_shape=out_shapes,
        scratch_shapes=[
            pltpu.VMEM((TOKENS, M), jnp.float32),
            pltpu.VMEM((BLOCK, M), jnp.float32),
            pltpu.VMEM((1, M), jnp.float32),
        ],
        compiler_params=pltpu.CompilerParams(
            dimension_semantics=("arbitrary", "arbitrary"),
        ),
    )(x, K, V, s2)
    return (y, f.reshape(TOKENS), g.reshape(TOKENS), attn)


# speedup vs baseline: 1.0398x; 1.0064x over previous
import jax
import jax.numpy as jnp
from jax.experimental import pallas as pl
from jax.experimental.pallas import tpu as pltpu

M = 64
D_IN = 1024
D_OUT = 1024
TOKENS = 8192
TOPK = 8
TAU = 0.1
BETA = 10.0
THETA = 0.5
GAMMA = 0.5

BLOCK = 2048
NB = TOKENS // BLOCK
NEG = -1e30


def _body(x_ref, k_ref, v_ref, s_ref, y_ref, attn_ref, f_ref, g_ref,
          s_scr, acc_scr, cnt_scr):
    p = pl.program_id(0)
    i = pl.program_id(1)

    @pl.when(p == 0)
    def phase0():
        xb = x_ref[...]
        xn = xb * (1.0 / (jnp.sqrt(jnp.sum(xb * xb, axis=1, keepdims=True))
                          + 1e-12))
        kb = k_ref[...]
        kn = kb * (1.0 / (jnp.sqrt(jnp.sum(kb * kb, axis=1, keepdims=True))
                          + 1e-12))
        sb = jax.lax.dot_general(xn, kn, (((1,), (1,)), ((), ())),
                                 preferred_element_type=jnp.float32)
        s_scr[pl.ds(i * BLOCK, BLOCK), :] = sb

        m = jnp.max(sb, axis=1, keepdims=True)
        eq = (sb == m).astype(jnp.float32)

        @pl.when(i == 0)
        def _init():
            acc_scr[...] = eq

        @pl.when(i != 0)
        def _acc():
            acc_scr[...] += eq

    @pl.when(p == 1)
    def phase1():
        @pl.when(i == 0)
        def _reduce_counts():
            ones = jnp.ones((1, BLOCK), dtype=jnp.float32)
            cnt_scr[...] = jax.lax.dot_general(
                ones, acc_scr[...], (((1,), (0,)), ((), ())),
                preferred_element_type=jnp.float32)

        sb = s_scr[pl.ds(i * BLOCK, BLOCK), :]

        rr = jax.lax.broadcasted_iota(jnp.int32, (M, M), 0)
        cc = jax.lax.broadcasted_iota(jnp.int32, (M, M), 1)
        lt = (rr < cc).astype(jnp.float32)
        work = sb
        mask = jnp.zeros((BLOCK, M), dtype=jnp.bool_)
        f = None
        for t in range(TOPK):
            mx = jnp.max(work, axis=1, keepdims=True)
            if t == 0:
                f = mx
            eq = (work == mx).astype(jnp.float32)
            pre = jax.lax.dot_general(eq, lt, (((1,), (0,)), ((), ())),
                                      preferred_element_type=jnp.float32)
            sel = jnp.logical_and(pre == 0.0, eq > 0.0)
            mask = jnp.logical_or(mask, sel)
            work = jnp.where(sel, NEG, work)

        counts = cnt_scr[...]
        bias = (-GAMMA / TOKENS) * counts
        logits = sb * (1.0 / TAU) + jnp.log(s_ref[...] + 1e-08) + bias
        logits = jnp.where(mask, logits, NEG)
        mrow = jnp.max(logits, axis=1, keepdims=True)
        e = jnp.where(mask, jnp.exp(logits - mrow), 0.0)
        attn = e * (1.0 / jnp.sum(e, axis=1, keepdims=True))

        vr = jax.lax.dot_general(attn, v_ref[...], (((1,), (0,)), ((), ())),
                                 preferred_element_type=jnp.float32)
        g = 1.0 - jax.nn.sigmoid(BETA * (f - THETA))

        y_ref[...] = g * vr
        attn_ref[...] = attn
        f_ref[...] = f
        g_ref[...] = g


@jax.jit
def kernel(x, K, V, s):
    s2 = s.reshape(1, M)
    out_shapes = (
        jax.ShapeDtypeStruct((TOKENS, D_OUT), jnp.float32),
        jax.ShapeDtypeStruct((TOKENS, M), jnp.float32),
        jax.ShapeDtypeStruct((TOKENS, 1), jnp.float32),
        jax.ShapeDtypeStruct((TOKENS, 1), jnp.float32),
    )
    in_specs = [
        pl.BlockSpec((BLOCK, D_IN),
                     lambda p, i: (jax.lax.select(p == 0, i, NB - 1), 0)),
        pl.BlockSpec((M, D_IN), lambda p, i: (0, 0)),
        pl.BlockSpec((M, D_OUT), lambda p, i: (0, 0)),
        pl.BlockSpec((1, M), lambda p, i: (0, 0)),
    ]
    out_idx = lambda p, i: (jax.lax.select(p == 0, 0, i), 0)
    out_specs = (
        pl.BlockSpec((BLOCK, D_OUT), out_idx),
        pl.BlockSpec((BLOCK, M), out_idx),
        pl.BlockSpec((BLOCK, 1), out_idx),
        pl.BlockSpec((BLOCK, 1), out_idx),
    )
    y, attn, f, g = pl.pallas_call(
        _body,
        grid=(2, NB),
        in_specs=in_specs,
        out_specs=out_specs,
        out_shape=out_shapes,
        scratch_shapes=[
            pltpu.VMEM((TOKENS, M), jnp.float32),
            pltpu.VMEM((BLOCK, M), jnp.float32),
            pltpu.VMEM((1, M), jnp.float32),
        ],
        compiler_params=pltpu.CompilerParams(
            dimension_semantics=("arbitrary", "arbitrary"),
        ),
    )(x, K, V, s2)
    return (y, f.reshape(TOKENS), g.reshape(TOKENS), attn)
